# baseline (speedup 1.0000x reference)
import functools

import jax
import jax.numpy as jnp
from jax import lax
from jax.experimental import pallas as pl
from jax.experimental.pallas import tpu as pltpu
from jax.experimental.pallas import tpu_sc as plsc

NUM_ROWS = 8
DIM = 100
DIM_PAD = 128
REAL = 50
SUB = 56

_info = plsc.get_sparse_core_info()
_NC, _NS = _info.num_cores, _info.num_subcores
_NW = _NC * _NS


def _make_sc_gather(B: int, C: int):
    per_w = B // _NW
    n_chunks = per_w // C
    assert n_chunks % 2 == 0 and n_chunks * C == per_w
    mesh = plsc.VectorSubcoreMesh(core_axis_name="c", subcore_axis_name="s")

    nbuf = 4
    assert n_chunks % nbuf == 0

    @functools.partial(
        pl.kernel,
        mesh=mesh,
        out_type=jax.ShapeDtypeStruct((B, DIM_PAD), jnp.float32),
        scratch_types=[
            pltpu.VMEM((per_w,), jnp.int32),
        ] + [pltpu.VMEM((C, DIM_PAD), jnp.float32)] * nbuf
        + [pltpu.SemaphoreType.DMA] * (4 * nbuf),
    )
    def k(idx_hbm, table_hbm, out_hbm, idx_v, *bufs):
        rows = bufs[:nbuf]
        sg = (bufs[nbuf:2 * nbuf], bufs[2 * nbuf:3 * nbuf])
        sw = (bufs[3 * nbuf:4 * nbuf], bufs[4 * nbuf:])
        wid = lax.axis_index("s") * _NC + lax.axis_index("c")
        w0 = wid * per_w

        pltpu.sync_copy(idx_hbm.at[pl.ds(w0, per_w)], idx_v)

        def _gather(g, b, h):
            return pltpu.make_async_copy(
                table_hbm.at[idx_v.at[pl.ds(g * C + h * SUB, REAL)]],
                rows[b].at[pl.ds(h * SUB, REAL)], sg[h][b])

        def gather_start(g, b):
            _gather(g, b, 0).start()
            _gather(g, b, 1).start()

        def gather_wait(g, b):
            _gather(g, b, 0).wait()
            _gather(g, b, 1).wait()

        def wout_start(g, b):
            pltpu.async_copy(
                rows[b], out_hbm.at[pl.ds(w0 + g * C, C)], sw[0][b])

        def wout_wait(g, b):
            pltpu.make_async_copy(
                rows[b], out_hbm.at[pl.ds(w0 + g * C, C)], sw[0][b]).wait()

        for b in range(nbuf):
            gather_start(b, b)

        def body(i, carry):
            for b in range(nbuf):
                g = nbuf * i + b
                gather_wait(g, b)
                wout_start(g, b)
                @pl.when(i < n_chunks // nbuf - 1)
                def _():
                    wout_wait(g, b)
                    gather_start(g + nbuf, b)
            return carry

        lax.fori_loop(0, n_chunks // nbuf, body, 0)
        for b in range(nbuf):
            wout_wait(n_chunks - nbuf + b, b)

    return k


REPLICAS = 512


def kernel(input, table):
    n_outer, n_inner = input.shape
    idx = jnp.pad(input.astype(jnp.int32),
                  ((0, 0), (0, SUB - n_inner))).reshape(-1)
    table_pad = jnp.pad(table, ((0, 0), (0, DIM_PAD - DIM)))
    table_rep = jnp.broadcast_to(
        table_pad[None], (REPLICAS, NUM_ROWS, DIM_PAD)
    ).reshape(REPLICAS * NUM_ROWS, DIM_PAD)
    idx = idx + NUM_ROWS * (
        jnp.arange(idx.shape[0], dtype=jnp.int32) % REPLICAS)
    out = _make_sc_gather(idx.shape[0], 2 * SUB)(idx, table_rep)
    return out.reshape(n_outer, SUB, DIM_PAD)[:, :n_inner, :DIM]


# speedup vs baseline: 1.0218x; 1.0218x over previous
import functools

import jax
import jax.numpy as jnp
from jax import lax
from jax.experimental import pallas as pl
from jax.experimental.pallas import tpu as pltpu
from jax.experimental.pallas import tpu_sc as plsc

NUM_ROWS = 8
DIM = 100
DIM_PAD = 128
REAL = 50
SUB = 56

_info = plsc.get_sparse_core_info()
_NC, _NS = _info.num_cores, _info.num_subcores
_NW = _NC * _NS


def _make_sc_gather(B: int, C: int):
    per_w = B // _NW
    n_chunks = per_w // C
    assert n_chunks % 2 == 0 and n_chunks * C == per_w
    mesh = plsc.VectorSubcoreMesh(core_axis_name="c", subcore_axis_name="s")

    nbuf = 4
    assert n_chunks % nbuf == 0

    @functools.partial(
        pl.kernel,
        mesh=mesh,
        out_type=jax.ShapeDtypeStruct((B, DIM_PAD), jnp.float32),
        scratch_types=[
            pltpu.VMEM((per_w,), jnp.int32),
        ] + [pltpu.VMEM((C, DIM_PAD), jnp.float32)] * nbuf
        + [pltpu.SemaphoreType.DMA] * (4 * nbuf),
    )
    def k(idx_hbm, table_hbm, out_hbm, idx_v, *bufs):
        rows = bufs[:nbuf]
        sg = (bufs[nbuf:2 * nbuf], bufs[2 * nbuf:3 * nbuf])
        sw = (bufs[3 * nbuf:4 * nbuf], bufs[4 * nbuf:])
        wid = lax.axis_index("s") * _NC + lax.axis_index("c")
        w0 = wid * per_w

        pltpu.sync_copy(idx_hbm.at[pl.ds(w0, per_w)], idx_v)

        def _gather(g, b, h):
            return pltpu.make_async_copy(
                table_hbm.at[idx_v.at[pl.ds(g * C + h * SUB, REAL)]],
                rows[b].at[pl.ds(h * SUB, REAL)], sg[h][b])

        def gather_start(g, b):
            _gather(g, b, 0).start()
            _gather(g, b, 1).start()

        def gather_wait(g, b):
            _gather(g, b, 0).wait()
            _gather(g, b, 1).wait()

        def wout_start(g, b):
            pltpu.async_copy(
                rows[b], out_hbm.at[pl.ds(w0 + g * C, C)], sw[0][b])

        def wout_wait(g, b):
            pltpu.make_async_copy(
                rows[b], out_hbm.at[pl.ds(w0 + g * C, C)], sw[0][b]).wait()

        for b in range(nbuf):
            gather_start(b, b)

        def body(i, carry):
            for b in range(nbuf):
                g = nbuf * i + b
                gather_wait(g, b)
                wout_start(g, b)
                @pl.when(i < n_chunks // nbuf - 1)
                def _():
                    wout_wait(g, b)
                    gather_start(g + nbuf, b)
            return carry

        lax.fori_loop(0, n_chunks // nbuf, body, 0)
        for b in range(nbuf):
            wout_wait(n_chunks - nbuf + b, b)

    return k


REPLICAS = 1024


def kernel(input, table):
    n_outer, n_inner = input.shape
    idx = jnp.pad(input.astype(jnp.int32),
                  ((0, 0), (0, SUB - n_inner))).reshape(-1)
    table_pad = jnp.pad(table, ((0, 0), (0, DIM_PAD - DIM)))
    table_rep = jnp.broadcast_to(
        table_pad[None], (REPLICAS, NUM_ROWS, DIM_PAD)
    ).reshape(REPLICAS * NUM_ROWS, DIM_PAD)
    idx = idx + NUM_ROWS * (
        jnp.arange(idx.shape[0], dtype=jnp.int32) % REPLICAS)
    out = _make_sc_gather(idx.shape[0], 2 * SUB)(idx, table_rep)
    return out.reshape(n_outer, SUB, DIM_PAD)[:, :n_inner, :DIM]
